# baseline (speedup 1.0000x reference)
import functools

import jax
import jax.numpy as jnp
from jax import lax
from jax.experimental import pallas as pl
from jax.experimental.pallas import tpu as pltpu
from jax.experimental.pallas import tpu_sc as plsc

NUM_ROWS = 10
EMBED_DIM = 128
BATCH = 16384

_NC = 2
_NS = 16
_NW = _NC * _NS
_BPW = BATCH // _NW
_CHUNK = 128
_NCHUNK = _BPW // _CHUNK


@functools.partial(
    pl.kernel,
    out_type=jax.ShapeDtypeStruct((BATCH, EMBED_DIM), jnp.float32),
    mesh=plsc.VectorSubcoreMesh(core_axis_name="c", subcore_axis_name="s"),
    scratch_types=[
        pltpu.VMEM((_NCHUNK, _CHUNK), jnp.int32),
        pltpu.VMEM((_BPW, EMBED_DIM), jnp.float32),
        pltpu.SemaphoreType.DMA,
    ],
)
def _lookup(idx_hbm, table_hbm, out_hbm, idx_v, rows_v, sem):
    wid = lax.axis_index("s") * _NC + lax.axis_index("c")
    base = wid * _BPW
    pltpu.sync_copy(idx_hbm.at[wid], idx_v)
    copies = [
        pltpu.async_copy(
            table_hbm.at[idx_v.at[j]],
            rows_v.at[pl.ds(j * _CHUNK, _CHUNK)],
            sem,
        )
        for j in range(_NCHUNK)
    ]
    for c in copies:
        c.wait()
    pltpu.sync_copy(rows_v, out_hbm.at[pl.ds(base, _BPW)])


def kernel(tags, table):
    idx = tags.astype(jnp.int32).reshape(_NW, _NCHUNK, _CHUNK)
    return _lookup(idx, table)


# speedup vs baseline: 2.2302x; 2.2302x over previous
import functools

import jax
import jax.numpy as jnp
from jax import lax
from jax.experimental import pallas as pl
from jax.experimental.pallas import tpu as pltpu
from jax.experimental.pallas import tpu_sc as plsc

NUM_ROWS = 10
EMBED_DIM = 128
BATCH = 16384

_NC = 2
_NS = 16
_NW = _NC * _NS
_BPW = BATCH // _NW
_CHUNK = 128
_NCHUNK = _BPW // _CHUNK
_L = 16
_ROWS_PER_STEP = 16


@functools.partial(
    pl.kernel,
    out_type=jax.ShapeDtypeStruct((BATCH, EMBED_DIM), jnp.float32),
    mesh=plsc.VectorSubcoreMesh(core_axis_name="c", subcore_axis_name="s"),
    scratch_types=[
        pltpu.VMEM((_BPW,), jnp.int32),
        pltpu.VMEM((NUM_ROWS * EMBED_DIM,), jnp.float32),
        pltpu.VMEM((_BPW, EMBED_DIM), jnp.float32),
        pltpu.SemaphoreType.DMA,
    ],
)
def _lookup(idx_hbm, table_hbm, out_hbm, idx_v, table_v, out_v, sem):
    wid = lax.axis_index("s") * _NC + lax.axis_index("c")
    base = wid * _BPW
    pltpu.sync_copy(idx_hbm.at[wid], idx_v)
    pltpu.sync_copy(table_hbm, table_v)
    copies = []
    for chunk in range(_NCHUNK):
        def body(it, carry, chunk=chunk):
            i0 = chunk * _CHUNK + it * _ROWS_PER_STEP
            vidx = idx_v[pl.ds(i0, _ROWS_PER_STEP)]
            for u in range(_ROWS_PER_STEP):
                i = i0 + u
                r = vidx[u]
                rbase = r * EMBED_DIM
                for c in range(EMBED_DIM // _L):
                    out_v[i, pl.ds(c * _L, _L)] = table_v[
                        pl.ds(rbase + c * _L, _L)
                    ]
            return carry

        lax.fori_loop(0, _CHUNK // _ROWS_PER_STEP, body, 0)
        copies.append(
            pltpu.async_copy(
                out_v.at[pl.ds(chunk * _CHUNK, _CHUNK)],
                out_hbm.at[pl.ds(base + chunk * _CHUNK, _CHUNK)],
                sem,
            )
        )
    for c in copies:
        c.wait()


def kernel(tags, table):
    idx = tags.astype(jnp.int32).reshape(_NW, _BPW)
    return _lookup(idx, table.reshape(-1))
